# baseline (speedup 1.0000x reference)
import jax
import jax.numpy as jnp
from jax import lax
from jax.experimental import pallas as pl
from jax.experimental.pallas import tpu as pltpu
from jax.experimental.pallas import tpu_sc as plsc

N = 8192
A = 8
D = 256
NC = 2
BLOCK = 32
ROWS_PER_CORE = N // NC


def _sc_tile(e):
    mesh = plsc.ScalarSubcoreMesh(axis_name="core", num_cores=NC)

    @pl.kernel(
        out_type=jax.ShapeDtypeStruct((N, D), jnp.float32),
        mesh=mesh,
        scratch_types=[pltpu.VMEM_SHARED((BLOCK, D), jnp.float32),
                       pltpu.SemaphoreType.DMA],
    )
    def k(e_hbm, o_hbm, shared, sem):
        cid = lax.axis_index("core")
        fills = [
            pltpu.async_copy(e_hbm, shared.at[pl.ds(s * A, A)], sem)
            for s in range(BLOCK // A)
        ]
        for c in fills:
            c.wait()
        writes = [
            pltpu.async_copy(
                shared,
                o_hbm.at[pl.ds(cid * ROWS_PER_CORE + j * BLOCK, BLOCK)],
                sem)
            for j in range(ROWS_PER_CORE // BLOCK)
        ]
        for c in writes:
            c.wait()

    return k(e)


def kernel(x, E_relative_position):
    del x
    return _sc_tile(E_relative_position)
